# Initial kernel scaffold; baseline (speedup 1.0000x reference)
import jax
import jax.numpy as jnp
from jax.experimental import pallas as pl


def kernel(x, edge_index, batch, W, b):
    raise NotImplementedError("write your pallas kernel here")


# speedup vs baseline: 97.2544x; 97.2544x over previous
import functools

import jax
import jax.numpy as jnp
from jax import lax
from jax.experimental import pallas as pl
from jax.experimental.pallas import tpu as pltpu
from jax.experimental.pallas import tpu_sc as plsc

N_WORKERS = 32
LANES = 16


def _matvec_body(x_ref, w_ref, b_ref, o_ref):
    o_ref[...] = lax.dot_general(
        w_ref[...], x_ref[...],
        dimension_numbers=(((1,), (1,)), ((), ())),
        preferred_element_type=jnp.float32,
    ) + b_ref[0]


def _elevation(x, W, b):
    n = x.shape[0]
    return pl.pallas_call(
        _matvec_body,
        out_shape=jax.ShapeDtypeStruct((1, n), jnp.float32),
        in_specs=[
            pl.BlockSpec(memory_space=pltpu.VMEM),
            pl.BlockSpec(memory_space=pltpu.VMEM),
            pl.BlockSpec(memory_space=pltpu.SMEM),
        ],
        out_specs=pl.BlockSpec(memory_space=pltpu.VMEM),
    )(x, W, b)


@functools.lru_cache(maxsize=None)
def _make_edge_accum(n, e_total):
    assert e_total % N_WORKERS == 0 and n % LANES == 0
    e_per = e_total // N_WORKERS
    assert e_per % LANES == 0
    n_vec = e_per // LANES
    mesh = plsc.VectorSubcoreMesh(core_axis_name="c", subcore_axis_name="s")

    @functools.partial(
        pl.kernel,
        mesh=mesh,
        out_type=jax.ShapeDtypeStruct((4 * N_WORKERS, n), jnp.float32),
        compiler_params=pltpu.CompilerParams(needs_layout_passes=False),
        scratch_types=[
            pltpu.VMEM((n,), jnp.float32),
            pltpu.VMEM((e_per,), jnp.int32),
            pltpu.VMEM((e_per,), jnp.int32),
            pltpu.VMEM((n,), jnp.float32),
            pltpu.VMEM((n,), jnp.float32),
            pltpu.VMEM((n,), jnp.float32),
            pltpu.VMEM((n,), jnp.float32),
        ],
    )
    def edge_accum(e_hbm, src_hbm, dst_hbm, out_hbm,
                   e_v, src_v, dst_v, cnt_v, sum_v, gt_v, lt_v):
        c = lax.axis_index("c")
        s = lax.axis_index("s")
        wid = s * 2 + c
        base = wid * e_per
        pltpu.sync_copy(e_hbm, e_v)
        pltpu.sync_copy(src_hbm.at[pl.ds(base, e_per)], src_v)
        pltpu.sync_copy(dst_hbm.at[pl.ds(base, e_per)], dst_v)

        zeros16 = jnp.zeros((LANES,), jnp.float32)
        ones16 = jnp.ones((LANES,), jnp.float32)

        def zero_body(i, carry):
            sl = pl.ds(i * LANES, LANES)
            cnt_v[sl] = zeros16
            sum_v[sl] = zeros16
            gt_v[sl] = zeros16
            lt_v[sl] = zeros16
            return carry

        lax.fori_loop(0, n // LANES, zero_body, 0)

        def edge_body(i, carry):
            sl = pl.ds(i * LANES, LANES)
            sv = src_v[sl]
            dv = dst_v[sl]
            es = plsc.load_gather(e_v, [sv])
            ed = plsc.load_gather(e_v, [dv])
            plsc.addupdate_scatter(cnt_v, [dv], ones16)
            plsc.addupdate_scatter(sum_v, [dv], es)
            plsc.addupdate_scatter(gt_v, [dv],
                                   jnp.where(ed >= es, ones16, zeros16))
            plsc.addupdate_scatter(lt_v, [dv],
                                   jnp.where(ed <= es, ones16, zeros16))
            return carry

        lax.fori_loop(0, n_vec, edge_body, 0)

        pltpu.sync_copy(cnt_v, out_hbm.at[wid])
        pltpu.sync_copy(sum_v, out_hbm.at[N_WORKERS + wid])
        pltpu.sync_copy(gt_v, out_hbm.at[2 * N_WORKERS + wid])
        pltpu.sync_copy(lt_v, out_hbm.at[3 * N_WORKERS + wid])

    return edge_accum


def _epilogue_body(p_ref, e_ref, sm_ref, pk_ref, tr_ref):
    w = N_WORKERS
    cnt = jnp.sum(p_ref[0:w, :], axis=0, keepdims=True)
    ssum = jnp.sum(p_ref[w:2 * w, :], axis=0, keepdims=True)
    gt = jnp.sum(p_ref[2 * w:3 * w, :], axis=0, keepdims=True)
    lt = jnp.sum(p_ref[3 * w:4 * w, :], axis=0, keepdims=True)
    e = e_ref[...]
    sm_ref[...] = (ssum + e) / (cnt + 1.0)
    pk_ref[...] = jnp.where(gt == cnt, 1.0, 0.0).astype(jnp.float32)
    tr_ref[...] = jnp.where(lt == cnt, 1.0, 0.0).astype(jnp.float32)


def _epilogue(partials, e_row):
    n = e_row.shape[1]
    return pl.pallas_call(
        _epilogue_body,
        out_shape=(
            jax.ShapeDtypeStruct((1, n), jnp.float32),
            jax.ShapeDtypeStruct((1, n), jnp.float32),
            jax.ShapeDtypeStruct((1, n), jnp.float32),
        ),
    )(partials, e_row)


def kernel(x, edge_index, batch, W, b):
    n = x.shape[0]
    e_total = edge_index.shape[1]
    ei = edge_index.astype(jnp.int32)
    src = ei[0]
    dst = ei[1]

    e_row = _elevation(x, W, b)
    e = e_row.reshape(n)

    partials = _make_edge_accum(n, e_total)(e, src, dst)

    sm, pk, tr = _epilogue(partials, e_row)
    return (sm.reshape(n),
            pk.reshape(n).astype(bool),
            tr.reshape(n).astype(bool))


# speedup vs baseline: 112.4000x; 1.1557x over previous
import functools

import jax
import jax.numpy as jnp
from jax import lax
from jax.experimental import pallas as pl
from jax.experimental.pallas import tpu as pltpu
from jax.experimental.pallas import tpu_sc as plsc

N_WORKERS = 32
LANES = 16


def _matvec_body(x_ref, w_ref, b_ref, o_ref):
    o_ref[...] = lax.dot_general(
        w_ref[...], x_ref[...],
        dimension_numbers=(((1,), (1,)), ((), ())),
        preferred_element_type=jnp.float32,
    ) + b_ref[0]


def _elevation(x, W, b):
    n = x.shape[0]
    return pl.pallas_call(
        _matvec_body,
        out_shape=jax.ShapeDtypeStruct((1, n), jnp.float32),
        in_specs=[
            pl.BlockSpec(memory_space=pltpu.VMEM),
            pl.BlockSpec(memory_space=pltpu.VMEM),
            pl.BlockSpec(memory_space=pltpu.SMEM),
        ],
        out_specs=pl.BlockSpec(memory_space=pltpu.VMEM),
    )(x, W, b)


@functools.lru_cache(maxsize=None)
def _make_edge_accum(n, e_total):
    assert e_total % N_WORKERS == 0 and n % LANES == 0
    e_per = e_total // N_WORKERS
    assert e_per % LANES == 0
    n_vec = e_per // LANES
    mesh = plsc.VectorSubcoreMesh(core_axis_name="c", subcore_axis_name="s")

    @functools.partial(
        pl.kernel,
        mesh=mesh,
        out_type=(
            jax.ShapeDtypeStruct((N_WORKERS, n), jnp.int32),
            jax.ShapeDtypeStruct((2 * N_WORKERS, n), jnp.float32),
        ),
        compiler_params=pltpu.CompilerParams(needs_layout_passes=False),
        scratch_types=[
            pltpu.VMEM((n,), jnp.float32),
            pltpu.VMEM((e_per,), jnp.int32),
            pltpu.VMEM((e_per,), jnp.int32),
            pltpu.VMEM((n,), jnp.int32),
            pltpu.VMEM((n,), jnp.float32),
            pltpu.VMEM((n,), jnp.float32),
            pltpu.SemaphoreType.DMA,
        ],
    )
    def edge_accum(e_hbm, src_hbm, dst_hbm, a_hbm, f_hbm,
                   e_v, src_v, dst_v, a_v, sum_v, eq_v, sem):
        c = lax.axis_index("c")
        s = lax.axis_index("s")
        wid = s * 2 + c
        base = wid * e_per
        cp_e = pltpu.async_copy(e_hbm, e_v, sem)
        cp_s = pltpu.async_copy(src_hbm.at[pl.ds(base, e_per)], src_v, sem)
        cp_d = pltpu.async_copy(dst_hbm.at[pl.ds(base, e_per)], dst_v, sem)

        zeros16f = jnp.zeros((LANES,), jnp.float32)
        zeros16i = jnp.zeros((LANES,), jnp.int32)

        @plsc.parallel_loop(0, n // LANES, unroll=8)
        def _(i):
            sl = pl.ds(i * LANES, LANES)
            a_v[sl] = zeros16i
            sum_v[sl] = zeros16f
            eq_v[sl] = zeros16f

        cp_e.wait()
        cp_s.wait()
        cp_d.wait()

        ones16f = jnp.ones((LANES,), jnp.float32)
        packed16 = jnp.full((LANES,), (1 << 14) + 1, jnp.int32)
        ones16i = jnp.ones((LANES,), jnp.int32)

        @plsc.parallel_loop(0, n_vec, unroll=5)
        def _(i):
            sl = pl.ds(i * LANES, LANES)
            sv = src_v[sl]
            dv = dst_v[sl]
            es = plsc.load_gather(e_v, [sv])
            ed = plsc.load_gather(e_v, [dv])
            plsc.addupdate_scatter(
                a_v, [dv], jnp.where(ed >= es, packed16, ones16i))
            plsc.addupdate_scatter(sum_v, [dv], es)
            plsc.addupdate_scatter(
                eq_v, [dv], jnp.where(ed == es, ones16f, zeros16f))

        pltpu.sync_copy(a_v, a_hbm.at[wid])
        pltpu.sync_copy(sum_v, f_hbm.at[wid])
        pltpu.sync_copy(eq_v, f_hbm.at[N_WORKERS + wid])

    return edge_accum


def _epilogue_body(a_ref, f_ref, e_ref, sm_ref, pk_ref, tr_ref):
    w = N_WORKERS
    a = a_ref[...]
    cnt_i = jnp.sum(a & ((1 << 14) - 1), axis=0, keepdims=True)
    gt_i = jnp.sum(a >> 14, axis=0, keepdims=True)
    cnt = cnt_i.astype(jnp.float32)
    gt = gt_i.astype(jnp.float32)
    ssum = jnp.sum(f_ref[0:w, :], axis=0, keepdims=True)
    eq = jnp.sum(f_ref[w:2 * w, :], axis=0, keepdims=True)
    lt = cnt - gt + eq
    e = e_ref[...]
    sm_ref[...] = (ssum + e) / (cnt + 1.0)
    pk_ref[...] = jnp.where(gt == cnt, 1.0, 0.0).astype(jnp.float32)
    tr_ref[...] = jnp.where(lt == cnt, 1.0, 0.0).astype(jnp.float32)


def _epilogue(a_part, f_part, e_row):
    n = e_row.shape[1]
    return pl.pallas_call(
        _epilogue_body,
        out_shape=(
            jax.ShapeDtypeStruct((1, n), jnp.float32),
            jax.ShapeDtypeStruct((1, n), jnp.float32),
            jax.ShapeDtypeStruct((1, n), jnp.float32),
        ),
    )(a_part, f_part, e_row)


def kernel(x, edge_index, batch, W, b):
    n = x.shape[0]
    e_total = edge_index.shape[1]
    ei = edge_index.astype(jnp.int32)
    src = ei[0]
    dst = ei[1]

    e_row = _elevation(x, W, b)
    e = e_row.reshape(n)

    a_part, f_part = _make_edge_accum(n, e_total)(e, src, dst)

    sm, pk, tr = _epilogue(a_part, f_part, e_row)
    return (sm.reshape(n),
            pk.reshape(n).astype(bool),
            tr.reshape(n).astype(bool))


# speedup vs baseline: 138.1212x; 1.2288x over previous
import functools

import jax
import jax.numpy as jnp
from jax import lax
from jax.experimental import pallas as pl
from jax.experimental.pallas import tpu as pltpu
from jax.experimental.pallas import tpu_sc as plsc

N_WORKERS = 32
LANES = 16


def _matvec_body(x_ref, w_ref, b_ref, o_ref):
    o_ref[...] = lax.dot_general(
        w_ref[...], x_ref[...],
        dimension_numbers=(((1,), (1,)), ((), ())),
        preferred_element_type=jnp.float32,
    ) + b_ref[0]


def _elevation(x, W, b):
    n = x.shape[0]
    return pl.pallas_call(
        _matvec_body,
        out_shape=jax.ShapeDtypeStruct((1, n), jnp.float32),
        in_specs=[
            pl.BlockSpec(memory_space=pltpu.VMEM),
            pl.BlockSpec(memory_space=pltpu.VMEM),
            pl.BlockSpec(memory_space=pltpu.SMEM),
        ],
        out_specs=pl.BlockSpec(memory_space=pltpu.VMEM),
    )(x, W, b)


@functools.lru_cache(maxsize=None)
def _make_edge_accum(n, e_total):
    assert e_total % N_WORKERS == 0 and n % LANES == 0
    e_per = e_total // N_WORKERS
    assert e_per % LANES == 0
    n_vec = e_per // LANES
    mesh = plsc.VectorSubcoreMesh(core_axis_name="c", subcore_axis_name="s")

    @functools.partial(
        pl.kernel,
        mesh=mesh,
        out_type=(
            jax.ShapeDtypeStruct((N_WORKERS, n), jnp.int32),
            jax.ShapeDtypeStruct((2 * N_WORKERS, n), jnp.float32),
        ),
        compiler_params=pltpu.CompilerParams(needs_layout_passes=False),
        scratch_types=[
            pltpu.VMEM((n,), jnp.float32),
            pltpu.VMEM((e_per,), jnp.int32),
            pltpu.VMEM((e_per,), jnp.int32),
            pltpu.VMEM((n,), jnp.int32),
            pltpu.VMEM((n,), jnp.float32),
            pltpu.VMEM((n,), jnp.float32),
            pltpu.SemaphoreType.DMA,
        ],
    )
    def edge_accum(e_hbm, ei_hbm, a_hbm, f_hbm,
                   e_v, src_v, dst_v, a_v, sum_v, eq_v, sem):
        c = lax.axis_index("c")
        s = lax.axis_index("s")
        wid = s * 2 + c
        base = wid * e_per
        cp_e = pltpu.async_copy(e_hbm, e_v, sem)
        cp_s = pltpu.async_copy(ei_hbm.at[pl.ds(base, e_per)], src_v, sem)
        cp_d = pltpu.async_copy(
            ei_hbm.at[pl.ds(e_total + base, e_per)], dst_v, sem)

        zeros16f = jnp.zeros((LANES,), jnp.float32)
        zeros16i = jnp.zeros((LANES,), jnp.int32)

        @plsc.parallel_loop(0, n // LANES, unroll=8)
        def _(i):
            sl = pl.ds(i * LANES, LANES)
            a_v[sl] = zeros16i
            sum_v[sl] = zeros16f
            eq_v[sl] = zeros16f

        cp_e.wait()
        cp_s.wait()
        cp_d.wait()

        ones16f = jnp.ones((LANES,), jnp.float32)
        packed16 = jnp.full((LANES,), (1 << 14) + 1, jnp.int32)
        ones16i = jnp.ones((LANES,), jnp.int32)

        @plsc.parallel_loop(0, n_vec, unroll=5)
        def _(i):
            sl = pl.ds(i * LANES, LANES)
            sv = src_v[sl]
            dv = dst_v[sl]
            es = plsc.load_gather(e_v, [sv])
            ed = plsc.load_gather(e_v, [dv])
            plsc.addupdate_scatter(
                a_v, [dv], jnp.where(ed >= es, packed16, ones16i))
            plsc.addupdate_scatter(sum_v, [dv], es)
            plsc.addupdate_scatter(
                eq_v, [dv], jnp.where(ed == es, ones16f, zeros16f))

        pltpu.sync_copy(a_v, a_hbm.at[wid])
        pltpu.sync_copy(sum_v, f_hbm.at[wid])
        pltpu.sync_copy(eq_v, f_hbm.at[N_WORKERS + wid])

    return edge_accum


def _epilogue_body(a_ref, f_ref, e_ref, sm_ref, pk_ref, tr_ref):
    w = N_WORKERS
    a = a_ref[...]
    cnt_i = jnp.sum(a & ((1 << 14) - 1), axis=0, keepdims=True)
    gt_i = jnp.sum(a >> 14, axis=0, keepdims=True)
    cnt = cnt_i.astype(jnp.float32)
    gt = gt_i.astype(jnp.float32)
    ssum = jnp.sum(f_ref[0:w, :], axis=0, keepdims=True)
    eq = jnp.sum(f_ref[w:2 * w, :], axis=0, keepdims=True)
    lt = cnt - gt + eq
    e = e_ref[...]
    sm_ref[...] = (ssum + e) / (cnt + 1.0)
    pk_ref[...] = gt == cnt
    tr_ref[...] = lt == cnt


def _epilogue(a_part, f_part, e_row):
    n = e_row.shape[1]
    return pl.pallas_call(
        _epilogue_body,
        out_shape=(
            jax.ShapeDtypeStruct((1, n), jnp.float32),
            jax.ShapeDtypeStruct((1, n), jnp.bool_),
            jax.ShapeDtypeStruct((1, n), jnp.bool_),
        ),
    )(a_part, f_part, e_row)


def kernel(x, edge_index, batch, W, b):
    n = x.shape[0]
    e_total = edge_index.shape[1]
    ei_flat = edge_index.astype(jnp.int32).reshape(2 * e_total)

    e_row = _elevation(x, W, b)

    a_part, f_part = _make_edge_accum(n, e_total)(e_row.reshape(n), ei_flat)

    sm, pk, tr = _epilogue(a_part, f_part, e_row)
    return (sm.reshape(n), pk.reshape(n), tr.reshape(n))
